# Initial kernel scaffold; baseline (speedup 1.0000x reference)
import jax
import jax.numpy as jnp
from jax.experimental import pallas as pl


def kernel(ego_embed, edge_index, edge_type, relation_embed):
    raise NotImplementedError("write your pallas kernel here")


# speedup vs baseline: 2.9542x; 2.9542x over previous
import functools

import jax
import jax.numpy as jnp
from jax import lax
from jax.experimental import pallas as pl
from jax.experimental.pallas import tpu as pltpu
from jax.experimental.pallas import tpu_sc as plsc

EPS = 1e-7
MAX_NORM = 1.0 - 1e-5

_SC_PARAMS = pltpu.CompilerParams(use_tc_tiling_on_sc=False)

NC = 2
NS = 16
NW = NC * NS
GCH = 80
L = 16


def _norm(x):
    return jnp.sqrt(jnp.clip(jnp.sum(x * x, axis=-1, keepdims=True), EPS, None))


def _artanh(x):
    x = jnp.clip(x, -1.0 + EPS, 1.0 - EPS)
    return 0.5 * (jnp.log1p(x) - jnp.log1p(-x))


def _mobius_add(x, y):
    x2 = jnp.sum(x * x, axis=-1, keepdims=True)
    y2 = jnp.sum(y * y, axis=-1, keepdims=True)
    xy = jnp.sum(x * y, axis=-1, keepdims=True)
    num = (1.0 + 2.0 * xy + y2) * x + (1.0 - x2) * y
    den = 1.0 + 2.0 * xy + x2 * y2
    return num / jnp.maximum(den, EPS)


def _project(x):
    n = _norm(x)
    scale = jnp.where(n > MAX_NORM, MAX_NORM / n, 1.0)
    return x * scale


def _expmap0(u):
    n = _norm(u)
    return _project(jnp.tanh(n) * u / n)


def _lambda(p):
    return 2.0 / jnp.clip(1.0 - jnp.sum(p * p, axis=-1, keepdims=True), EPS, None)


def _expmap(u, p):
    n = _norm(u)
    second = jnp.tanh(0.5 * _lambda(p) * n) * u / n
    return _project(_mobius_add(p, second))


def _logmap(x, p):
    sub = _mobius_add(-p, x)
    n = _norm(sub)
    return (2.0 / _lambda(p)) * _artanh(n) * sub / n


def _make_gather(N, E, D):
    epw = E // NW
    nch = epw // GCH
    mesh = plsc.VectorSubcoreMesh(core_axis_name="c", subcore_axis_name="s")

    @functools.partial(
        pl.kernel,
        out_type=(
            jax.ShapeDtypeStruct((E, D), jnp.float32),
            jax.ShapeDtypeStruct((E, D), jnp.float32),
        ),
        mesh=mesh,
        scratch_types=[
            pltpu.VMEM((GCH,), jnp.int32),
            pltpu.VMEM((GCH, D), jnp.float32),
            pltpu.SemaphoreType.DMA,
        ],
        compiler_params=_SC_PARAMS,
    )
    def gather_k(ego_hbm, heads_hbm, tails_hbm, hout, tout, idx_v, rows_v, sem):
        wid = lax.axis_index("s") * NC + lax.axis_index("c")
        base = pl.multiple_of(wid * epw, 8)

        def run(idx_hbm, out_hbm):
            @pl.loop(0, nch)
            def _(ci):
                off = pl.multiple_of(base + ci * GCH, 8)
                pltpu.sync_copy(idx_hbm.at[pl.ds(off, GCH)], idx_v)
                pltpu.async_copy(ego_hbm.at[idx_v], rows_v, sem).wait()
                pltpu.sync_copy(rows_v, out_hbm.at[pl.ds(off, GCH)])

        run(heads_hbm, hout)
        run(tails_hbm, tout)

    return gather_k


def _make_scatter(N, E, D):
    hn = N // NC
    acc = hn + 8
    eps_ = E // NS
    nch = eps_ // GCH
    rps = (acc // NS) // 8 * 8
    tail = acc - NS * rps
    stg = rps + tail
    dtail = hn - NS * rps
    mesh = plsc.VectorSubcoreMesh(core_axis_name="c", subcore_axis_name="s")

    @functools.partial(
        pl.kernel,
        out_type=(
            jax.ShapeDtypeStruct((N, D), jnp.float32),
            jax.ShapeDtypeStruct((N, 16), jnp.float32),
        ),
        mesh=mesh,
        scratch_types=[
            pltpu.VMEM((GCH,), jnp.int32),
            pltpu.VMEM((GCH, D), jnp.float32),
            pltpu.VMEM((GCH, 16), jnp.float32),
            pltpu.VMEM((stg, D), jnp.float32),
            pltpu.VMEM((stg, 16), jnp.float32),
            pltpu.VMEM_SHARED((acc, D), jnp.float32),
            pltpu.VMEM_SHARED((acc, 16), jnp.float32),
        ],
        compiler_params=_SC_PARAMS,
    )
    def scatter_k(res_hbm, heads_hbm, zsum_hbm, zcnt_hbm, ones_hbm,
                  osum, ocnt, idx_v, rows_v, ones_v, stg_s, stg_c, ssum, scnt):
        cid = lax.axis_index("c")
        sid = lax.axis_index("s")
        base = pl.multiple_of(sid * eps_, 8)
        row0 = sid * rps
        t0 = NS * rps
        nlo = cid * hn

        pltpu.sync_copy(zsum_hbm, stg_s)
        pltpu.sync_copy(zcnt_hbm, stg_c)
        pltpu.sync_copy(stg_s.at[pl.ds(0, rps)], ssum.at[pl.ds(row0, rps)])
        pltpu.sync_copy(stg_c.at[pl.ds(0, rps)], scnt.at[pl.ds(row0, rps)])

        @pl.when(sid == NS - 1)
        def _():
            pltpu.sync_copy(stg_s.at[pl.ds(rps, tail)], ssum.at[pl.ds(t0, tail)])
            pltpu.sync_copy(stg_c.at[pl.ds(rps, tail)], scnt.at[pl.ds(t0, tail)])

        pltpu.sync_copy(ones_hbm, ones_v)
        plsc.subcore_barrier()

        @pl.loop(0, nch)
        def _(ci):
            off = pl.multiple_of(base + ci * GCH, 8)
            pltpu.sync_copy(heads_hbm.at[pl.ds(off, GCH)], idx_v)
            pltpu.sync_copy(res_hbm.at[pl.ds(off, GCH)], rows_v)

            @pl.loop(0, GCH // L)
            def _(j):
                v = idx_v[pl.ds(j * L, L)] - nlo
                ok = (v >= 0) & (v < hn)
                idx_v[pl.ds(j * L, L)] = jnp.where(ok, v, hn)

            pltpu.sync_copy(rows_v, ssum.at[idx_v], add=True)
            pltpu.sync_copy(ones_v, scnt.at[idx_v], add=True)

        plsc.subcore_barrier()
        pltpu.sync_copy(ssum.at[pl.ds(row0, rps)], stg_s.at[pl.ds(0, rps)])
        pltpu.sync_copy(scnt.at[pl.ds(row0, rps)], stg_c.at[pl.ds(0, rps)])

        @pl.when(sid == NS - 1)
        def _():
            pltpu.sync_copy(ssum.at[pl.ds(t0, dtail)], stg_s.at[pl.ds(rps, dtail)])
            pltpu.sync_copy(scnt.at[pl.ds(t0, dtail)], stg_c.at[pl.ds(rps, dtail)])
            pltpu.sync_copy(stg_s.at[pl.ds(0, rps + dtail)],
                            osum.at[pl.ds(nlo + row0, rps + dtail)])
            pltpu.sync_copy(stg_c.at[pl.ds(0, rps + dtail)],
                            ocnt.at[pl.ds(nlo + row0, rps + dtail)])

        @pl.when(sid != NS - 1)
        def _():
            pltpu.sync_copy(stg_s.at[pl.ds(0, rps)],
                            osum.at[pl.ds(nlo + row0, rps)])
            pltpu.sync_copy(stg_c.at[pl.ds(0, rps)],
                            ocnt.at[pl.ds(nlo + row0, rps)])

    return scatter_k


def _edge_math_body(h_ref, t_ref, et_ref, rel_ref, o_ref):
    h = h_ref[...]
    t = t_ref[...]
    et = et_ref[...]
    rel_tab = rel_ref[...]
    R = rel_tab.shape[0]
    onehot = (et == lax.broadcasted_iota(jnp.int32, (1, R), 1)).astype(jnp.float32)
    r = jnp.dot(onehot, rel_tab, preferred_element_type=jnp.float32)
    p = _expmap0(h)
    ht = _expmap(t, p)
    hr = _expmap(r, p)
    res = _project(_mobius_add(ht, hr))
    o_ref[...] = _logmap(res, p)


def _divide_body(s_ref, c_ref, o_ref):
    o_ref[...] = s_ref[...] / jnp.maximum(c_ref[:, 0:1], 1.0)


@jax.jit
def _run(ego_embed, edge_index, edge_type, relation_embed):
    N, D = ego_embed.shape
    E = edge_index.shape[1]
    R = relation_embed.shape[0]
    heads = edge_index[0]
    tails = edge_index[1]

    h_emb, t_emb = _make_gather(N, E, D)(ego_embed, heads, tails)

    EB = 2560
    neb = E // EB
    res = pl.pallas_call(
        _edge_math_body,
        grid=(neb,),
        in_specs=[
            pl.BlockSpec((EB, D), lambda i: (i, 0)),
            pl.BlockSpec((EB, D), lambda i: (i, 0)),
            pl.BlockSpec((EB, 1), lambda i: (i, 0)),
            pl.BlockSpec((R, D), lambda i: (0, 0)),
        ],
        out_specs=pl.BlockSpec((EB, D), lambda i: (i, 0)),
        out_shape=jax.ShapeDtypeStruct((E, D), jnp.float32),
    )(h_emb, t_emb, edge_type.reshape(E, 1), relation_embed)

    acc = N // NC + 8
    rps = (acc // NS) // 8 * 8
    stg = rps + (acc - NS * rps)
    zsum = jnp.zeros((stg, D), jnp.float32)
    zcnt = jnp.zeros((stg, 16), jnp.float32)
    ones = jnp.ones((GCH, 16), jnp.float32)
    sums, cnts = _make_scatter(N, E, D)(res, heads, zsum, zcnt, ones)

    NB = 2000
    out = pl.pallas_call(
        _divide_body,
        grid=(N // NB,),
        in_specs=[
            pl.BlockSpec((NB, D), lambda i: (i, 0)),
            pl.BlockSpec((NB, 16), lambda i: (i, 0)),
        ],
        out_specs=pl.BlockSpec((NB, D), lambda i: (i, 0)),
        out_shape=jax.ShapeDtypeStruct((N, D), jnp.float32),
    )(sums, cnts)
    return out


def kernel(ego_embed, edge_index, edge_type, relation_embed):
    return _run(ego_embed, edge_index, edge_type, relation_embed)


# speedup vs baseline: 3.2663x; 1.1056x over previous
import functools

import jax
import jax.numpy as jnp
from jax import lax
from jax.experimental import pallas as pl
from jax.experimental.pallas import tpu as pltpu
from jax.experimental.pallas import tpu_sc as plsc

EPS = 1e-7
MAX_NORM = 1.0 - 1e-5

_SC_PARAMS = pltpu.CompilerParams(use_tc_tiling_on_sc=False)

NC = 2
NS = 16
NW = NC * NS
GCH = 80
L = 16


def _norm(x):
    return jnp.sqrt(jnp.clip(jnp.sum(x * x, axis=-1, keepdims=True), EPS, None))


def _artanh(x):
    x = jnp.clip(x, -1.0 + EPS, 1.0 - EPS)
    return 0.5 * (jnp.log1p(x) - jnp.log1p(-x))


def _mobius_add(x, y):
    x2 = jnp.sum(x * x, axis=-1, keepdims=True)
    y2 = jnp.sum(y * y, axis=-1, keepdims=True)
    xy = jnp.sum(x * y, axis=-1, keepdims=True)
    num = (1.0 + 2.0 * xy + y2) * x + (1.0 - x2) * y
    den = 1.0 + 2.0 * xy + x2 * y2
    return num / jnp.maximum(den, EPS)


def _project(x):
    n = _norm(x)
    scale = jnp.where(n > MAX_NORM, MAX_NORM / n, 1.0)
    return x * scale


def _expmap0(u):
    n = _norm(u)
    return _project(jnp.tanh(n) * u / n)


def _lambda(p):
    return 2.0 / jnp.clip(1.0 - jnp.sum(p * p, axis=-1, keepdims=True), EPS, None)


def _expmap(u, p):
    n = _norm(u)
    second = jnp.tanh(0.5 * _lambda(p) * n) * u / n
    return _project(_mobius_add(p, second))


def _logmap(x, p):
    sub = _mobius_add(-p, x)
    n = _norm(sub)
    return (2.0 / _lambda(p)) * _artanh(n) * sub / n


NSUB = 5
KCH = NSUB * GCH


def _make_gather(N, E, D):
    epw = E // NW
    nmac = epw // KCH
    mesh = plsc.VectorSubcoreMesh(core_axis_name="c", subcore_axis_name="s")

    @functools.partial(
        pl.kernel,
        out_type=(
            jax.ShapeDtypeStruct((E, D), jnp.float32),
            jax.ShapeDtypeStruct((E, D), jnp.float32),
        ),
        mesh=mesh,
        scratch_types=[
            pltpu.VMEM((NSUB, GCH), jnp.int32),
            pltpu.VMEM((KCH, D), jnp.float32),
            pltpu.SemaphoreType.DMA,
        ],
        compiler_params=_SC_PARAMS,
    )
    def gather_k(ego_hbm, heads2_hbm, tails2_hbm, hout, tout, idx_v, rows_v, sem):
        wid = lax.axis_index("s") * NC + lax.axis_index("c")
        base = pl.multiple_of(wid * epw, 8)
        crow0 = wid * (epw // GCH)

        def run(idx2_hbm, out_hbm):
            @pl.loop(0, nmac)
            def _(ci):
                off = pl.multiple_of(base + ci * KCH, 8)
                pltpu.sync_copy(idx2_hbm.at[pl.ds(crow0 + ci * NSUB, NSUB)],
                                idx_v)
                copies = [
                    pltpu.async_copy(ego_hbm.at[idx_v.at[j]],
                                     rows_v.at[pl.ds(j * GCH, GCH)], sem)
                    for j in range(NSUB)
                ]
                for c in copies:
                    c.wait()
                pltpu.sync_copy(rows_v, out_hbm.at[pl.ds(off, KCH)])

        run(heads2_hbm, hout)
        run(tails2_hbm, tout)

    return gather_k


def _make_scatter(N, E, D):
    hn = N // NC
    acc = hn + 8
    eps_ = E // NS
    nmac = eps_ // KCH
    rps = (acc // NS) // 8 * 8
    tail = acc - NS * rps
    stg = rps + tail
    dtail = hn - NS * rps
    mesh = plsc.VectorSubcoreMesh(core_axis_name="c", subcore_axis_name="s")

    @functools.partial(
        pl.kernel,
        out_type=(
            jax.ShapeDtypeStruct((N, D), jnp.float32),
            jax.ShapeDtypeStruct((N, 16), jnp.float32),
        ),
        mesh=mesh,
        scratch_types=[
            pltpu.VMEM((GCH,), jnp.int32),
            pltpu.VMEM((GCH, D), jnp.float32),
            pltpu.VMEM((GCH, 16), jnp.float32),
            pltpu.VMEM((stg, D), jnp.float32),
            pltpu.VMEM((stg, 16), jnp.float32),
            pltpu.VMEM_SHARED((acc, D), jnp.float32),
            pltpu.VMEM_SHARED((acc, 16), jnp.float32),
        ],
        compiler_params=_SC_PARAMS,
    )
    def scatter_k(res_hbm, heads_hbm, zsum_hbm, zcnt_hbm, ones_hbm,
                  osum, ocnt, idx_v, rows_v, ones_v, stg_s, stg_c, ssum, scnt):
        cid = lax.axis_index("c")
        sid = lax.axis_index("s")
        base = pl.multiple_of(sid * eps_, 8)
        row0 = sid * rps
        t0 = NS * rps
        nlo = cid * hn

        pltpu.sync_copy(zsum_hbm, stg_s)
        pltpu.sync_copy(zcnt_hbm, stg_c)
        pltpu.sync_copy(stg_s.at[pl.ds(0, rps)], ssum.at[pl.ds(row0, rps)])
        pltpu.sync_copy(stg_c.at[pl.ds(0, rps)], scnt.at[pl.ds(row0, rps)])

        @pl.when(sid == NS - 1)
        def _():
            pltpu.sync_copy(stg_s.at[pl.ds(rps, tail)], ssum.at[pl.ds(t0, tail)])
            pltpu.sync_copy(stg_c.at[pl.ds(rps, tail)], scnt.at[pl.ds(t0, tail)])

        pltpu.sync_copy(ones_hbm, ones_v)
        plsc.subcore_barrier()

        @pl.loop(0, nmac * NSUB)
        def _(ci):
            off = pl.multiple_of(base + ci * GCH, 8)
            pltpu.sync_copy(heads_hbm.at[pl.ds(off, GCH)], idx_v)
            pltpu.sync_copy(res_hbm.at[pl.ds(off, GCH)], rows_v)

            @pl.loop(0, GCH // L)
            def _(j):
                v = idx_v[pl.ds(j * L, L)] - nlo
                ok = (v >= 0) & (v < hn)
                idx_v[pl.ds(j * L, L)] = jnp.where(ok, v, hn)

            pltpu.sync_copy(rows_v, ssum.at[idx_v], add=True)
            pltpu.sync_copy(ones_v, scnt.at[idx_v], add=True)

        plsc.subcore_barrier()
        pltpu.sync_copy(ssum.at[pl.ds(row0, rps)], stg_s.at[pl.ds(0, rps)])
        pltpu.sync_copy(scnt.at[pl.ds(row0, rps)], stg_c.at[pl.ds(0, rps)])

        @pl.when(sid == NS - 1)
        def _():
            pltpu.sync_copy(ssum.at[pl.ds(t0, dtail)], stg_s.at[pl.ds(rps, dtail)])
            pltpu.sync_copy(scnt.at[pl.ds(t0, dtail)], stg_c.at[pl.ds(rps, dtail)])
            pltpu.sync_copy(stg_s.at[pl.ds(0, rps + dtail)],
                            osum.at[pl.ds(nlo + row0, rps + dtail)])
            pltpu.sync_copy(stg_c.at[pl.ds(0, rps + dtail)],
                            ocnt.at[pl.ds(nlo + row0, rps + dtail)])

        @pl.when(sid != NS - 1)
        def _():
            pltpu.sync_copy(stg_s.at[pl.ds(0, rps)],
                            osum.at[pl.ds(nlo + row0, rps)])
            pltpu.sync_copy(stg_c.at[pl.ds(0, rps)],
                            ocnt.at[pl.ds(nlo + row0, rps)])

    return scatter_k


def _edge_math_body(h_ref, t_ref, et_ref, rel_ref, o_ref):
    h = h_ref[...]
    t = t_ref[...]
    et = et_ref[...]
    rel_tab = rel_ref[...]
    R = rel_tab.shape[0]
    onehot = (et == lax.broadcasted_iota(jnp.int32, (1, R), 1)).astype(jnp.float32)
    r = jnp.dot(onehot, rel_tab, preferred_element_type=jnp.float32)
    p = _expmap0(h)
    ht = _expmap(t, p)
    hr = _expmap(r, p)
    res = _project(_mobius_add(ht, hr))
    o_ref[...] = _logmap(res, p)


def _divide_body(s_ref, c_ref, o_ref):
    o_ref[...] = s_ref[...] / jnp.maximum(c_ref[:, 0:1], 1.0)


@jax.jit
def _run(ego_embed, edge_index, edge_type, relation_embed):
    N, D = ego_embed.shape
    E = edge_index.shape[1]
    R = relation_embed.shape[0]
    heads = edge_index[0]
    tails = edge_index[1]
    heads2 = heads.reshape(E // GCH, GCH)
    tails2 = tails.reshape(E // GCH, GCH)

    h_emb, t_emb = _make_gather(N, E, D)(ego_embed, heads2, tails2)

    EB = 2560
    neb = E // EB
    res = pl.pallas_call(
        _edge_math_body,
        grid=(neb,),
        in_specs=[
            pl.BlockSpec((EB, D), lambda i: (i, 0)),
            pl.BlockSpec((EB, D), lambda i: (i, 0)),
            pl.BlockSpec((EB, 1), lambda i: (i, 0)),
            pl.BlockSpec((R, D), lambda i: (0, 0)),
        ],
        out_specs=pl.BlockSpec((EB, D), lambda i: (i, 0)),
        out_shape=jax.ShapeDtypeStruct((E, D), jnp.float32),
    )(h_emb, t_emb, edge_type.reshape(E, 1), relation_embed)

    acc = N // NC + 8
    rps = (acc // NS) // 8 * 8
    stg = rps + (acc - NS * rps)
    zsum = jnp.zeros((stg, D), jnp.float32)
    zcnt = jnp.zeros((stg, 16), jnp.float32)
    ones = jnp.ones((GCH, 16), jnp.float32)
    sums, cnts = _make_scatter(N, E, D)(res, heads, zsum, zcnt, ones)

    NB = 2000
    out = pl.pallas_call(
        _divide_body,
        grid=(N // NB,),
        in_specs=[
            pl.BlockSpec((NB, D), lambda i: (i, 0)),
            pl.BlockSpec((NB, 16), lambda i: (i, 0)),
        ],
        out_specs=pl.BlockSpec((NB, D), lambda i: (i, 0)),
        out_shape=jax.ShapeDtypeStruct((N, D), jnp.float32),
    )(sums, cnts)
    return out


def kernel(ego_embed, edge_index, edge_type, relation_embed):
    return _run(ego_embed, edge_index, edge_type, relation_embed)
